# baseline (speedup 1.0000x reference)
import jax
import jax.numpy as jnp
from jax import lax
from jax.experimental import pallas as pl
from jax.experimental.pallas import tpu as pltpu
from jax.experimental.pallas import tpu_sc as plsc

N = 512
NC = 2
NS = 16
L = 16
NW = NC * NS
CHUNK = N // NW


def _sc_body(x_hbm, map_hbm, out_idx_hbm, out_val_hbm,
             xsh, x2, mine_map, rank_v, sem):
    s = lax.axis_index("s")
    w = s * NC + lax.axis_index("c")
    base = w * CHUNK

    @pl.when(s == 0)
    def _():
        pltpu.sync_copy(x_hbm, xsh)
    plsc.subcore_barrier()
    pltpu.sync_copy(xsh, x2.at[pl.ds(0, N)])
    pltpu.sync_copy(xsh.at[pl.ds(0, CHUNK)], x2.at[pl.ds(N, CHUNK)])
    pltpu.sync_copy(map_hbm.at[pl.ds(base, CHUNK)], mine_map)

    iota = lax.iota(jnp.int32, L)
    gidx = iota + base
    m = x2[pl.ds(base, L)]

    def step(k, carry):
        rank, jvec = carry
        wnd = x2[pl.ds(k, L)]
        jw = jvec & (N - 1)
        before = (wnd > m) | ((wnd == m) & (jw < gidx))
        return rank + jnp.where(before, 1, 0), jvec + 1

    rank, _ = lax.fori_loop(
        0, N, step, (jnp.zeros((L,), jnp.int32), iota), unroll=8
    )
    rank_v[...] = rank

    pltpu.async_copy(x2.at[pl.ds(base, CHUNK)], out_val_hbm.at[rank_v], sem
                     ).wait()
    pltpu.async_copy(mine_map, out_idx_hbm.at[rank_v], sem).wait()


@jax.jit
def _sc_sort(x, map_arr):
    call = pl.kernel(
        _sc_body,
        out_type=(
            jax.ShapeDtypeStruct((N,), jnp.int32),
            jax.ShapeDtypeStruct((N,), jnp.float32),
        ),
        mesh=plsc.VectorSubcoreMesh(core_axis_name="c", subcore_axis_name="s"),
        compiler_params=pltpu.CompilerParams(needs_layout_passes=False),
        scratch_types=(
            pltpu.VMEM_SHARED((N,), jnp.float32),
            pltpu.VMEM((N + CHUNK,), jnp.float32),
            pltpu.VMEM((CHUNK,), jnp.int32),
            pltpu.VMEM((CHUNK,), jnp.int32),
            pltpu.SemaphoreType.DMA,
        ),
    )
    return call(x, map_arr)


def kernel(input, map_arr):
    return _sc_sort(input, map_arr)


# speedup vs baseline: 1.4395x; 1.4395x over previous
import jax
import jax.numpy as jnp
from jax import lax
from jax.experimental import pallas as pl
from jax.experimental.pallas import tpu as pltpu
from jax.experimental.pallas import tpu_sc as plsc

N = 512
NS = 16
L = 16
CHUNK = N // NS


def _sc_body(x_hbm, map_hbm, out_idx_hbm, out_val_hbm,
             xsh, msh, ovsh, oish, x2, mine_map, rank_v, sem):
    s = lax.axis_index("s")
    base = s * CHUNK

    @pl.when(s == 0)
    def _():
        pltpu.sync_copy(x_hbm, xsh)
        pltpu.sync_copy(map_hbm, msh)
    plsc.subcore_barrier()
    pltpu.sync_copy(xsh, x2.at[pl.ds(0, N)])
    pltpu.sync_copy(xsh.at[pl.ds(0, L)], x2.at[pl.ds(N, L)])
    pltpu.sync_copy(msh.at[pl.ds(base, CHUNK)], mine_map)

    iota = lax.iota(jnp.int32, L)
    g1 = iota + base
    g2 = g1 + L
    m1 = x2[pl.ds(base, L)]
    m2 = x2[pl.ds(base + L, L)]

    def step(k, carry):
        r1, r2, jvec = carry
        wnd = x2[pl.ds(k, L)]
        jw = jvec & (N - 1)
        b1 = (wnd > m1) | ((wnd == m1) & (jw < g1))
        b2 = (wnd > m2) | ((wnd == m2) & (jw < g2))
        return (r1 + jnp.where(b1, 1, 0), r2 + jnp.where(b2, 1, 0),
                jvec + 1)

    zeros = jnp.zeros((L,), jnp.int32)
    r1, r2, _ = lax.fori_loop(0, N, step, (zeros, zeros, iota), unroll=8)
    rank_v[pl.ds(0, L)] = r1
    rank_v[pl.ds(L, L)] = r2

    pltpu.async_copy(x2.at[pl.ds(base, CHUNK)], ovsh.at[rank_v], sem).wait()
    pltpu.async_copy(mine_map, oish.at[rank_v], sem).wait()
    plsc.subcore_barrier()

    @pl.when(s == 0)
    def _():
        pltpu.sync_copy(ovsh, out_val_hbm)
        pltpu.sync_copy(oish, out_idx_hbm)


@jax.jit
def _sc_sort(x, map_arr):
    call = pl.kernel(
        _sc_body,
        out_type=(
            jax.ShapeDtypeStruct((N,), jnp.int32),
            jax.ShapeDtypeStruct((N,), jnp.float32),
        ),
        mesh=plsc.VectorSubcoreMesh(
            core_axis_name="c", subcore_axis_name="s", num_cores=1
        ),
        compiler_params=pltpu.CompilerParams(needs_layout_passes=False),
        scratch_types=(
            pltpu.VMEM_SHARED((N,), jnp.float32),
            pltpu.VMEM_SHARED((N,), jnp.int32),
            pltpu.VMEM_SHARED((N,), jnp.float32),
            pltpu.VMEM_SHARED((N,), jnp.int32),
            pltpu.VMEM((N + L,), jnp.float32),
            pltpu.VMEM((CHUNK,), jnp.int32),
            pltpu.VMEM((CHUNK,), jnp.int32),
            pltpu.SemaphoreType.DMA,
        ),
    )
    return call(x, map_arr)


def kernel(input, map_arr):
    return _sc_sort(input, map_arr)


# speedup vs baseline: 1.9838x; 1.3781x over previous
import jax
import jax.numpy as jnp
from jax import lax
from jax.experimental import pallas as pl
from jax.experimental.pallas import tpu as pltpu
from jax.experimental.pallas import tpu_sc as plsc

N = 512
NS = 16
L = 16
CHUNK = N // NS


def _sc_body(x_hbm, map_hbm, out_idx_hbm, out_val_hbm,
             ovsh, oish, x2, mine_map, rank_v, sem, sem2):
    s = lax.axis_index("s")
    base = s * CHUNK

    mcp = pltpu.async_copy(map_hbm.at[pl.ds(base, CHUNK)], mine_map, sem2)
    cp1 = pltpu.async_copy(x_hbm, x2.at[pl.ds(0, N)], sem)
    cp2 = pltpu.async_copy(x_hbm.at[pl.ds(0, L)], x2.at[pl.ds(N, L)], sem)
    cp1.wait()
    cp2.wait()

    iota = lax.iota(jnp.int32, L)
    g1 = iota + base
    g2 = g1 + L
    m1 = x2[pl.ds(base, L)]
    m2 = x2[pl.ds(base + L, L)]

    def step(k, carry):
        r1, r2, jvec = carry
        wnd = x2[pl.ds(k, L)]
        jw = jvec & (N - 1)
        b1 = (wnd > m1) | ((wnd == m1) & (jw < g1))
        b2 = (wnd > m2) | ((wnd == m2) & (jw < g2))
        return (r1 + jnp.where(b1, 1, 0), r2 + jnp.where(b2, 1, 0),
                jvec + 1)

    zeros = jnp.zeros((L,), jnp.int32)
    r1, r2, _ = lax.fori_loop(0, N, step, (zeros, zeros, iota), unroll=4)
    rank_v[pl.ds(0, L)] = r1
    rank_v[pl.ds(L, L)] = r2
    mcp.wait()

    sc1 = pltpu.async_copy(x2.at[pl.ds(base, CHUNK)], ovsh.at[rank_v], sem)
    sc2 = pltpu.async_copy(mine_map, oish.at[rank_v], sem)
    sc1.wait()
    sc2.wait()
    plsc.subcore_barrier()

    @pl.when(s == 0)
    def _():
        wb1 = pltpu.async_copy(ovsh, out_val_hbm, sem)
        wb2 = pltpu.async_copy(oish, out_idx_hbm, sem)
        wb1.wait()
        wb2.wait()


@jax.jit
def _sc_sort(x, map_arr):
    call = pl.kernel(
        _sc_body,
        out_type=(
            jax.ShapeDtypeStruct((N,), jnp.int32),
            jax.ShapeDtypeStruct((N,), jnp.float32),
        ),
        mesh=plsc.VectorSubcoreMesh(
            core_axis_name="c", subcore_axis_name="s", num_cores=1
        ),
        compiler_params=pltpu.CompilerParams(needs_layout_passes=False),
        scratch_types=(
            pltpu.VMEM_SHARED((N,), jnp.float32),
            pltpu.VMEM_SHARED((N,), jnp.int32),
            pltpu.VMEM((N + L,), jnp.float32),
            pltpu.VMEM((CHUNK,), jnp.int32),
            pltpu.VMEM((CHUNK,), jnp.int32),
            pltpu.SemaphoreType.DMA,
            pltpu.SemaphoreType.DMA,
        ),
    )
    return call(x, map_arr)


def kernel(input, map_arr):
    return _sc_sort(input, map_arr)


# speedup vs baseline: 1.9908x; 1.0035x over previous
import jax
import jax.numpy as jnp
from jax import lax
from jax.experimental import pallas as pl
from jax.experimental.pallas import tpu as pltpu
from jax.experimental.pallas import tpu_sc as plsc

N = 512
NS = 16
L = 16
CHUNK = N // NS


def _sc_body(x_hbm, map_hbm, out_idx_hbm, out_val_hbm,
             ovsh, oish, x2, mine_map, rank_v, sem, sem2):
    s = lax.axis_index("s")
    base = s * CHUNK

    mcp = pltpu.async_copy(map_hbm.at[pl.ds(base, CHUNK)], mine_map, sem2)
    pltpu.async_copy(x_hbm, x2.at[pl.ds(0, N)], sem).wait()
    x2[pl.ds(N, L)] = x2[pl.ds(0, L)]

    iota = lax.iota(jnp.int32, L)
    g1 = iota + base
    g2 = g1 + L
    m1 = x2[pl.ds(base, L)]
    m2 = x2[pl.ds(base + L, L)]

    def step(k, carry):
        r1, r2, jvec = carry
        wnd = x2[pl.ds(k, L)]
        jw = jvec & (N - 1)
        b1 = (wnd > m1) | ((wnd == m1) & (jw < g1))
        b2 = (wnd > m2) | ((wnd == m2) & (jw < g2))
        return (r1 + jnp.where(b1, 1, 0), r2 + jnp.where(b2, 1, 0),
                jvec + 1)

    zeros = jnp.zeros((L,), jnp.int32)
    r1, r2, _ = lax.fori_loop(0, N, step, (zeros, zeros, iota), unroll=4)
    rank_v[pl.ds(0, L)] = r1
    rank_v[pl.ds(L, L)] = r2
    mcp.wait()

    sc1 = pltpu.async_copy(x2.at[pl.ds(base, CHUNK)], ovsh.at[rank_v], sem)
    sc2 = pltpu.async_copy(mine_map, oish.at[rank_v], sem)
    sc1.wait()
    sc2.wait()
    plsc.subcore_barrier()

    @pl.when(s == 0)
    def _():
        wb1 = pltpu.async_copy(ovsh, out_val_hbm, sem)
        wb2 = pltpu.async_copy(oish, out_idx_hbm, sem)
        wb1.wait()
        wb2.wait()


@jax.jit
def _sc_sort(x, map_arr):
    call = pl.kernel(
        _sc_body,
        out_type=(
            jax.ShapeDtypeStruct((N,), jnp.int32),
            jax.ShapeDtypeStruct((N,), jnp.float32),
        ),
        mesh=plsc.VectorSubcoreMesh(
            core_axis_name="c", subcore_axis_name="s", num_cores=1
        ),
        compiler_params=pltpu.CompilerParams(needs_layout_passes=False),
        scratch_types=(
            pltpu.VMEM_SHARED((N,), jnp.float32),
            pltpu.VMEM_SHARED((N,), jnp.int32),
            pltpu.VMEM((N + L,), jnp.float32),
            pltpu.VMEM((CHUNK,), jnp.int32),
            pltpu.VMEM((CHUNK,), jnp.int32),
            pltpu.SemaphoreType.DMA,
            pltpu.SemaphoreType.DMA,
        ),
    )
    return call(x, map_arr)


def kernel(input, map_arr):
    return _sc_sort(input, map_arr)
